# baseline (speedup 1.0000x reference)
import functools

import jax
import jax.numpy as jnp
from jax import lax
from jax.experimental import pallas as pl
from jax.experimental.pallas import tpu as pltpu
from jax.experimental.pallas import tpu_sc as plsc

NC = 2
NS = 16
NW = NC * NS

V = 1000000
D = 64
T = 50
S = 16384
NB = 4

RU = 7812
R1 = RU // NW
GU = (T * (S // 128)) // NW


def _iota16():
    return lax.iota(jnp.int32, 16)


@jax.jit
def _relayout(tableT, tail128):
    mesh = plsc.VectorSubcoreMesh(core_axis_name="c", subcore_axis_name="s")

    @functools.partial(
        pl.kernel,
        mesh=mesh,
        out_type=jax.ShapeDtypeStruct((V // 2, 128), jnp.float32),
        scratch_types=[
            pltpu.VMEM((NB, 64, 128), jnp.float32),
            pltpu.VMEM((NB, 64, 128), jnp.float32),
            pltpu.SemaphoreType.DMA((NB,)),
            pltpu.SemaphoreType.DMA((NB,)),
        ],
        compiler_params=pltpu.CompilerParams(
            use_tc_tiling_on_sc=True, needs_layout_passes=False
        ),
    )
    def body(tT_hbm, tail_hbm, tlin_hbm, stage_v, outb_v, ssems, osems):
        wid = lax.axis_index("s") * NC + lax.axis_index("c")
        cvecs = [_iota16() + 16 * k for k in range(4)]

        def start_stage(b, r):
            u = r * NW + wid
            pltpu.async_copy(
                tT_hbm.at[:, pl.ds(128 * u, 128)], stage_v.at[b], ssems.at[b]
            )

        def wait_stage(b):
            pltpu.make_async_copy(
                tT_hbm.at[:, pl.ds(0, 128)], stage_v.at[b], ssems.at[b]
            ).wait()

        def start_out(b, r):
            u = r * NW + wid
            pltpu.async_copy(
                outb_v.at[b], tlin_hbm.at[pl.ds(64 * u, 64)], osems.at[b]
            )

        def wait_out(b):
            pltpu.make_async_copy(
                outb_v.at[b], tlin_hbm.at[pl.ds(0, 64)], osems.at[b]
            ).wait()

        def transpose(b, n_si):
            def blk(i8, si_vec):
                base = i8 * 8
                row = base // 2
                vecs = [si_vec + j for j in range(8)]
                for j in range(8):
                    for k in range(4):
                        v = plsc.load_gather(stage_v.at[b], [cvecs[k], vecs[j]])
                        outb_v[
                            b, row + j // 2, pl.ds((j % 2) * 64 + 16 * k, 16)
                        ] = v
                return si_vec + 8

            lax.fori_loop(0, n_si // 8, blk, jnp.zeros((16,), jnp.int32))

        for b in range(NB):
            start_stage(b, b)

        def step(q, b, first, last):
            r = q * NB + b
            wait_stage(b)
            if not first:
                wait_out(b)
            transpose(b, 128)
            if not last:
                start_stage(b, r + NB)
            start_out(b, r)

        for b in range(NB):
            step(0, b, True, False)

        def round_body(q, _):
            for b in range(NB):
                step(q, b, False, False)
            return _

        lax.fori_loop(1, R1 // NB - 1, round_body, None)

        for b in range(NB):
            step(R1 // NB - 1, b, False, True)
        for b in range(NB):
            wait_out(b)

        for k in range(NB):
            @pl.when(wid == k)
            def _(k=k):
                u = RU - NB + k
                pltpu.sync_copy(
                    tT_hbm.at[:, pl.ds(128 * u, 128)], stage_v.at[0]
                )
                transpose(0, 128)
                pltpu.sync_copy(outb_v.at[0], tlin_hbm.at[pl.ds(64 * u, 64)])

        @pl.when(wid == NB)
        def _():
            pltpu.sync_copy(tail_hbm, stage_v.at[0])
            transpose(0, 64)
            pltpu.sync_copy(
                outb_v.at[0, pl.ds(0, 32)],
                tlin_hbm.at[pl.ds(64 * RU, 32)],
            )

    return body(tableT, tail128)


@jax.jit
def _gather_t(xs, tl):
    mesh = plsc.VectorSubcoreMesh(core_axis_name="c", subcore_axis_name="s")

    @functools.partial(
        pl.kernel,
        mesh=mesh,
        out_type=jax.ShapeDtypeStruct((T, 8, S // 128, 8, 128), jnp.float32),
        scratch_types=[
            pltpu.VMEM((NB, 128), jnp.int32),
            pltpu.VMEM((NB, 128, 64), jnp.float32),
            pltpu.VMEM((NB, 8, 8, 128), jnp.float32),
            pltpu.SemaphoreType.DMA((NB,)),
            pltpu.SemaphoreType.DMA((NB,)),
            pltpu.SemaphoreType.DMA((NB,)),
        ],
        compiler_params=pltpu.CompilerParams(
            use_tc_tiling_on_sc=False, needs_layout_passes=False
        ),
    )
    def body(xs_hbm, tl_hbm, out_hbm, idx_v, rows_v, rowsT_v, isems, gsems, osems):
        wid = lax.axis_index("s") * NC + lax.axis_index("c")
        svecs = [_iota16() + 16 * k for k in range(8)]

        def unit(r):
            u = r * NW + wid
            return u // 128, u % 128

        def start_idx(b, r):
            t, w = unit(r)
            pltpu.async_copy(
                xs_hbm.at[t, pl.ds(128 * w, 128)], idx_v.at[b], isems.at[b]
            )

        def wait_idx(b):
            pltpu.make_async_copy(
                xs_hbm.at[0, pl.ds(0, 128)], idx_v.at[b], isems.at[b]
            ).wait()

        def start_gather(b):
            pltpu.async_copy(tl_hbm.at[idx_v.at[b]], rows_v.at[b], gsems.at[b])

        def wait_gather(b):
            pltpu.make_async_copy(
                tl_hbm.at[pl.ds(0, 128)], rows_v.at[b], gsems.at[b]
            ).wait()

        def start_out(b, r):
            t, w = unit(r)
            pltpu.async_copy(rowsT_v.at[b], out_hbm.at[t, :, w], osems.at[b])

        def wait_out(b):
            pltpu.make_async_copy(
                rowsT_v.at[b], out_hbm.at[0, :, 0], osems.at[b]
            ).wait()

        def transpose(b):
            def blk(i8, c_vec):
                vecs = [c_vec + j for j in range(4)]
                for j in range(4):
                    for k in range(8):
                        v = plsc.load_gather(rows_v.at[b], [svecs[k], vecs[j]])
                        rowsT_v[
                            b, i8 // 2, (i8 % 2) * 4 + j, pl.ds(16 * k, 16)
                        ] = v
                return c_vec + 4

            lax.fori_loop(0, 16, blk, jnp.zeros((16,), jnp.int32))

        for b in range(NB):
            start_idx(b, b)
        for b in range(NB):
            wait_idx(b)
            start_gather(b)

        def step(q, b, first, last):
            r = q * NB + b
            wait_gather(b)
            if not last:
                start_idx(b, r + NB)
            if not first:
                wait_out(b)
            transpose(b)
            start_out(b, r)
            if not last:
                wait_idx(b)
                start_gather(b)

        for b in range(NB):
            step(0, b, True, False)

        def round_body(q, _):
            for b in range(NB):
                step(q, b, False, False)
            return _

        lax.fori_loop(1, GU // NB - 1, round_body, None)

        for b in range(NB):
            step(GU // NB - 1, b, False, True)
        for b in range(NB):
            wait_out(b)

    return body(xs, tl)


def kernel(x, table):
    tableT = table.T
    xs = x.T.astype(jnp.int32)
    tail128 = jnp.pad(tableT[:, 128 * RU:], ((0, 0), (0, 64)))
    table_lin = _relayout(tableT, tail128)
    tl = table_lin.reshape(V, D)
    out5 = _gather_t(xs, tl)
    return out5.transpose(2, 4, 0, 1, 3).reshape(S, T, D)


# speedup vs baseline: 1.4380x; 1.4380x over previous
import functools

import jax
import jax.numpy as jnp
from jax import lax
from jax.experimental import pallas as pl
from jax.experimental.pallas import tpu as pltpu
from jax.experimental.pallas import tpu_sc as plsc

NC = 2
NS = 16
NW = NC * NS

V = 1000000
D = 64
T = 50
S = 16384
NB = 4

RU = 7812
R1 = RU // NW
GU = (T * (S // 128)) // NW


def _iota16():
    return lax.iota(jnp.int32, 16)


@jax.jit
def _relayout(tableT, tail128):
    mesh = plsc.VectorSubcoreMesh(core_axis_name="c", subcore_axis_name="s")

    @functools.partial(
        pl.kernel,
        mesh=mesh,
        out_type=jax.ShapeDtypeStruct((V // 2, 128), jnp.float32),
        scratch_types=[
            pltpu.VMEM((NB, 64, 129), jnp.float32),
            pltpu.VMEM((NB, 64, 128), jnp.float32),
            pltpu.SemaphoreType.DMA((NB,)),
            pltpu.SemaphoreType.DMA((NB,)),
        ],
        compiler_params=pltpu.CompilerParams(
            use_tc_tiling_on_sc=True, needs_layout_passes=False
        ),
    )
    def body(tT_hbm, tail_hbm, tlin_hbm, stage_v, outb_v, ssems, osems):
        wid = lax.axis_index("s") * NC + lax.axis_index("c")
        cvecs = [_iota16() + 16 * k for k in range(4)]

        def start_stage(b, r):
            u = r * NW + wid
            pltpu.async_copy(
                tT_hbm.at[:, pl.ds(128 * u, 128)],
                stage_v.at[b, :, pl.ds(0, 128)],
                ssems.at[b],
            )

        def wait_stage(b):
            pltpu.make_async_copy(
                tT_hbm.at[:, pl.ds(0, 128)],
                stage_v.at[b, :, pl.ds(0, 128)],
                ssems.at[b],
            ).wait()

        def start_out(b, r):
            u = r * NW + wid
            pltpu.async_copy(
                outb_v.at[b], tlin_hbm.at[pl.ds(64 * u, 64)], osems.at[b]
            )

        def wait_out(b):
            pltpu.make_async_copy(
                outb_v.at[b], tlin_hbm.at[pl.ds(0, 64)], osems.at[b]
            ).wait()

        def transpose(b, n_si):
            def blk(i8, si_vec):
                base = i8 * 8
                row = base // 2
                vecs = [si_vec + j for j in range(8)]
                for j in range(8):
                    for k in range(4):
                        v = plsc.load_gather(stage_v.at[b], [cvecs[k], vecs[j]])
                        outb_v[
                            b, row + j // 2, pl.ds((j % 2) * 64 + 16 * k, 16)
                        ] = v
                return si_vec + 8

            lax.fori_loop(0, n_si // 8, blk, jnp.zeros((16,), jnp.int32))

        for b in range(NB):
            start_stage(b, b)

        def step(q, b, first, last):
            r = q * NB + b
            wait_stage(b)
            if not first:
                wait_out(b)
            transpose(b, 128)
            if not last:
                start_stage(b, r + NB)
            start_out(b, r)

        for b in range(NB):
            step(0, b, True, False)

        def round_body(q, _):
            for b in range(NB):
                step(q, b, False, False)
            return _

        lax.fori_loop(1, R1 // NB - 1, round_body, None)

        for b in range(NB):
            step(R1 // NB - 1, b, False, True)
        for b in range(NB):
            wait_out(b)

        for k in range(NB):
            @pl.when(wid == k)
            def _(k=k):
                u = RU - NB + k
                pltpu.sync_copy(
                    tT_hbm.at[:, pl.ds(128 * u, 128)],
                    stage_v.at[0, :, pl.ds(0, 128)],
                )
                transpose(0, 128)
                pltpu.sync_copy(outb_v.at[0], tlin_hbm.at[pl.ds(64 * u, 64)])

        @pl.when(wid == NB)
        def _():
            pltpu.sync_copy(tail_hbm, stage_v.at[0, :, pl.ds(0, 128)])
            transpose(0, 64)
            pltpu.sync_copy(
                outb_v.at[0, pl.ds(0, 32)],
                tlin_hbm.at[pl.ds(64 * RU, 32)],
            )

    return body(tableT, tail128)


@jax.jit
def _gather_t(xs, tl):
    mesh = plsc.VectorSubcoreMesh(core_axis_name="c", subcore_axis_name="s")

    @functools.partial(
        pl.kernel,
        mesh=mesh,
        out_type=jax.ShapeDtypeStruct((T, 8, S // 128, 8, 128), jnp.float32),
        scratch_types=[
            pltpu.VMEM((NB, 128), jnp.int32),
            pltpu.VMEM((NB, 128, 64), jnp.float32),
            pltpu.VMEM((NB, 8, 8, 129), jnp.float32),
            pltpu.SemaphoreType.DMA((NB,)),
            pltpu.SemaphoreType.DMA((NB,)),
            pltpu.SemaphoreType.DMA((NB,)),
        ],
        compiler_params=pltpu.CompilerParams(
            use_tc_tiling_on_sc=False, needs_layout_passes=False
        ),
    )
    def body(xs_hbm, tl_hbm, out_hbm, idx_v, rows_v, rowsT_v, isems, gsems, osems):
        wid = lax.axis_index("s") * NC + lax.axis_index("c")

        def unit(r):
            u = r * NW + wid
            return u // 128, u % 128

        def start_idx(b, r):
            t, w = unit(r)
            pltpu.async_copy(
                xs_hbm.at[t, pl.ds(128 * w, 128)], idx_v.at[b], isems.at[b]
            )

        def wait_idx(b):
            pltpu.make_async_copy(
                xs_hbm.at[0, pl.ds(0, 128)], idx_v.at[b], isems.at[b]
            ).wait()

        def start_gather(b):
            pltpu.async_copy(tl_hbm.at[idx_v.at[b]], rows_v.at[b], gsems.at[b])

        def wait_gather(b):
            pltpu.make_async_copy(
                tl_hbm.at[pl.ds(0, 128)], rows_v.at[b], gsems.at[b]
            ).wait()

        def start_out(b, r):
            t, w = unit(r)
            pltpu.async_copy(
                rowsT_v.at[b, :, :, pl.ds(0, 128)],
                out_hbm.at[t, :, w],
                osems.at[b],
            )

        def wait_out(b):
            pltpu.make_async_copy(
                rowsT_v.at[b, :, :, pl.ds(0, 128)],
                out_hbm.at[0, :, 0],
                osems.at[b],
            ).wait()

        gvecs = [(_iota16() + 16 * k) // 8 for k in range(4)]
        civecs = [(_iota16() + 16 * k) % 8 for k in range(4)]

        def transpose(b):
            def blk(i8, si_vec):
                base = i8 * 8
                vecs = [si_vec + j for j in range(8)]
                for j in range(8):
                    for k in range(4):
                        v = rows_v[b, base + j, pl.ds(16 * k, 16)]
                        plsc.store_scatter(
                            rowsT_v.at[b], [gvecs[k], civecs[k], vecs[j]], v
                        )
                return si_vec + 8

            lax.fori_loop(0, 16, blk, jnp.zeros((16,), jnp.int32))

        for b in range(NB):
            start_idx(b, b)
        for b in range(NB):
            wait_idx(b)
            start_gather(b)

        def step(q, b, first, last):
            r = q * NB + b
            wait_gather(b)
            if not last:
                start_idx(b, r + NB)
            if not first:
                wait_out(b)
            transpose(b)
            start_out(b, r)
            if not last:
                wait_idx(b)
                start_gather(b)

        for b in range(NB):
            step(0, b, True, False)

        def round_body(q, _):
            for b in range(NB):
                step(q, b, False, False)
            return _

        lax.fori_loop(1, GU // NB - 1, round_body, None)

        for b in range(NB):
            step(GU // NB - 1, b, False, True)
        for b in range(NB):
            wait_out(b)

    return body(xs, tl)


def kernel(x, table):
    tableT = table.T
    xs = x.T.astype(jnp.int32)
    tail128 = jnp.pad(tableT[:, 128 * RU:], ((0, 0), (0, 64)))
    table_lin = _relayout(tableT, tail128)
    tl = table_lin.reshape(V, D)
    out5 = _gather_t(xs, tl)
    return out5.transpose(2, 4, 0, 1, 3).reshape(S, T, D)


# speedup vs baseline: 2.7686x; 1.9253x over previous
import functools

import jax
import jax.numpy as jnp
from jax import lax
from jax.experimental import pallas as pl
from jax.experimental.pallas import tpu as pltpu
from jax.experimental.pallas import tpu_sc as plsc

NC = 2
NS = 16
NW = NC * NS

V = 1000000
D = 64
T = 50
S = 16384
NB = 4

RU = 7812
R1 = RU // NW
GU = (T * (S // 128)) // NW


def _iota16():
    return lax.iota(jnp.int32, 16)


@jax.jit
def _relayout(tableT, tail128):
    mesh = plsc.VectorSubcoreMesh(core_axis_name="c", subcore_axis_name="s")

    @functools.partial(
        pl.kernel,
        mesh=mesh,
        out_type=jax.ShapeDtypeStruct((V // 2, 128), jnp.float32),
        scratch_types=[
            pltpu.VMEM((NB, 64, 128), jnp.float32),
            pltpu.VMEM((NB, 136, 65), jnp.float32),
            pltpu.SemaphoreType.DMA((NB,)),
            pltpu.SemaphoreType.DMA((NB,)),
        ],
        compiler_params=pltpu.CompilerParams(
            use_tc_tiling_on_sc=True, needs_layout_passes=False
        ),
    )
    def body(tT_hbm, tail_hbm, tlin_hbm, stage_v, outb_v, ssems, osems):
        wid = lax.axis_index("s") * NC + lax.axis_index("c")
        rvecs = [
            ((_iota16() + 16 * k) // 2) + 72 * ((_iota16() + 16 * k) % 2)
            for k in range(8)
        ]

        def start_stage(b, r):
            u = r * NW + wid
            pltpu.async_copy(
                tT_hbm.at[:, pl.ds(128 * u, 128)], stage_v.at[b], ssems.at[b]
            )

        def wait_stage(b):
            pltpu.make_async_copy(
                tT_hbm.at[:, pl.ds(0, 128)], stage_v.at[b], ssems.at[b]
            ).wait()

        def start_out(b, r):
            u = r * NW + wid
            pltpu.async_copy(
                outb_v.at[b, pl.ds(0, 64), pl.ds(0, 64)],
                tlin_hbm.at[pl.ds(64 * u, 64), pl.ds(0, 64)],
                osems.at[b],
            )
            pltpu.async_copy(
                outb_v.at[b, pl.ds(72, 64), pl.ds(0, 64)],
                tlin_hbm.at[pl.ds(64 * u, 64), pl.ds(64, 64)],
                osems.at[b],
            )

        def wait_out(b):
            pltpu.make_async_copy(
                outb_v.at[b, pl.ds(0, 64), pl.ds(0, 64)],
                tlin_hbm.at[pl.ds(0, 64), pl.ds(0, 64)],
                osems.at[b],
            ).wait()
            pltpu.make_async_copy(
                outb_v.at[b, pl.ds(72, 64), pl.ds(0, 64)],
                tlin_hbm.at[pl.ds(0, 64), pl.ds(64, 64)],
                osems.at[b],
            ).wait()

        def transpose(b, nk):
            def blk(i4, col_vec):
                base = i4 * 4
                cols = [col_vec + j for j in range(4)]
                for j in range(4):
                    for k in range(nk):
                        v = stage_v[b, base + j, pl.ds(16 * k, 16)]
                        plsc.store_scatter(
                            outb_v.at[b], [rvecs[k], cols[j]], v
                        )
                return col_vec + 4

            lax.fori_loop(0, 16, blk, jnp.zeros((16,), jnp.int32))

        for b in range(NB):
            start_stage(b, b)

        def step(q, b, first, last):
            r = q * NB + b
            wait_stage(b)
            if not first:
                wait_out(b)
            transpose(b, 8)
            if not last:
                start_stage(b, r + NB)
            start_out(b, r)

        for b in range(NB):
            step(0, b, True, False)

        def round_body(q, _):
            for b in range(NB):
                step(q, b, False, False)
            return _

        lax.fori_loop(1, R1 // NB - 1, round_body, None)

        for b in range(NB):
            step(R1 // NB - 1, b, False, True)
        for b in range(NB):
            wait_out(b)

        for k in range(NB):
            @pl.when(wid == k)
            def _(k=k):
                u = RU - NB + k
                pltpu.sync_copy(
                    tT_hbm.at[:, pl.ds(128 * u, 128)], stage_v.at[0]
                )
                transpose(0, 8)
                pltpu.sync_copy(
                    outb_v.at[0, pl.ds(0, 64), pl.ds(0, 64)],
                    tlin_hbm.at[pl.ds(64 * u, 64), pl.ds(0, 64)],
                )
                pltpu.sync_copy(
                    outb_v.at[0, pl.ds(72, 64), pl.ds(0, 64)],
                    tlin_hbm.at[pl.ds(64 * u, 64), pl.ds(64, 64)],
                )

        @pl.when(wid == NB)
        def _():
            pltpu.sync_copy(tail_hbm, stage_v.at[0])
            transpose(0, 4)
            pltpu.sync_copy(
                outb_v.at[0, pl.ds(0, 32), pl.ds(0, 64)],
                tlin_hbm.at[pl.ds(64 * RU, 32), pl.ds(0, 64)],
            )
            pltpu.sync_copy(
                outb_v.at[0, pl.ds(72, 32), pl.ds(0, 64)],
                tlin_hbm.at[pl.ds(64 * RU, 32), pl.ds(64, 64)],
            )

    return body(tableT, tail128)


@jax.jit
def _gather_t(xs, tl):
    mesh = plsc.VectorSubcoreMesh(core_axis_name="c", subcore_axis_name="s")

    @functools.partial(
        pl.kernel,
        mesh=mesh,
        out_type=jax.ShapeDtypeStruct((T, 8, S // 128, 8, 128), jnp.float32),
        scratch_types=[
            pltpu.VMEM((NB, 128), jnp.int32),
            pltpu.VMEM((NB, 128, 64), jnp.float32),
            pltpu.VMEM((NB, 8, 8, 129), jnp.float32),
            pltpu.SemaphoreType.DMA((NB,)),
            pltpu.SemaphoreType.DMA((NB,)),
            pltpu.SemaphoreType.DMA((NB,)),
        ],
        compiler_params=pltpu.CompilerParams(
            use_tc_tiling_on_sc=False, needs_layout_passes=False
        ),
    )
    def body(xs_hbm, tl_hbm, out_hbm, idx_v, rows_v, rowsT_v, isems, gsems, osems):
        wid = lax.axis_index("s") * NC + lax.axis_index("c")

        def unit(r):
            u = r * NW + wid
            return u // 128, u % 128

        def start_idx(b, r):
            t, w = unit(r)
            pltpu.async_copy(
                xs_hbm.at[t, pl.ds(128 * w, 128)], idx_v.at[b], isems.at[b]
            )

        def wait_idx(b):
            pltpu.make_async_copy(
                xs_hbm.at[0, pl.ds(0, 128)], idx_v.at[b], isems.at[b]
            ).wait()

        def start_gather(b):
            pltpu.async_copy(tl_hbm.at[idx_v.at[b]], rows_v.at[b], gsems.at[b])

        def wait_gather(b):
            pltpu.make_async_copy(
                tl_hbm.at[pl.ds(0, 128)], rows_v.at[b], gsems.at[b]
            ).wait()

        def start_out(b, r):
            t, w = unit(r)
            pltpu.async_copy(
                rowsT_v.at[b, :, :, pl.ds(0, 128)],
                out_hbm.at[t, :, w],
                osems.at[b],
            )

        def wait_out(b):
            pltpu.make_async_copy(
                rowsT_v.at[b, :, :, pl.ds(0, 128)],
                out_hbm.at[0, :, 0],
                osems.at[b],
            ).wait()

        gvecs = [(_iota16() + 16 * k) // 8 for k in range(4)]
        civecs = [(_iota16() + 16 * k) % 8 for k in range(4)]

        def transpose(b):
            def blk(i8, si_vec):
                base = i8 * 8
                vecs = [si_vec + j for j in range(8)]
                for j in range(8):
                    for k in range(4):
                        v = rows_v[b, base + j, pl.ds(16 * k, 16)]
                        plsc.store_scatter(
                            rowsT_v.at[b], [gvecs[k], civecs[k], vecs[j]], v
                        )
                return si_vec + 8

            lax.fori_loop(0, 16, blk, jnp.zeros((16,), jnp.int32))

        for b in range(NB):
            start_idx(b, b)
        for b in range(NB):
            wait_idx(b)
            start_gather(b)

        def step(q, b, first, last):
            r = q * NB + b
            wait_gather(b)
            if not last:
                start_idx(b, r + NB)
            if not first:
                wait_out(b)
            transpose(b)
            start_out(b, r)
            if not last:
                wait_idx(b)
                start_gather(b)

        for b in range(NB):
            step(0, b, True, False)

        def round_body(q, _):
            for b in range(NB):
                step(q, b, False, False)
            return _

        lax.fori_loop(1, GU // NB - 1, round_body, None)

        for b in range(NB):
            step(GU // NB - 1, b, False, True)
        for b in range(NB):
            wait_out(b)

    return body(xs, tl)


def kernel(x, table):
    xs = x.T.astype(jnp.int32)
    out5 = _gather_t(xs, table)
    return out5.transpose(2, 4, 0, 1, 3).reshape(S, T, D)
